# baseline (speedup 1.0000x reference)
import math

import jax
import jax.numpy as jnp
from jax.experimental import pallas as pl
from jax.experimental.pallas import tpu as pltpu

EMB_DIM = 1024
HALF_DIM = EMB_DIM // 2
PAD_IDX = 0
BLK = 1024


def _sinusoid_kernel(out_ref, base_sin, base_cos):
    pid = pl.program_id(0)
    scale = math.log(10000.0) / (HALF_DIM - 1)

    @pl.when(pid == 0)
    def _init():
        row = jax.lax.broadcasted_iota(jnp.int32, (BLK, HALF_DIM), 0)
        col = jax.lax.broadcasted_iota(jnp.int32, (BLK, HALF_DIM), 1)
        freq = jnp.exp(col.astype(jnp.float32) * jnp.float32(-scale))
        phase = (row.astype(jnp.float32) + jnp.float32(PAD_IDX + 1)) * freq
        base_sin[...] = jnp.sin(phase)
        base_cos[...] = jnp.cos(phase)

    colr = jax.lax.broadcasted_iota(jnp.int32, (8, HALF_DIM), 1).astype(jnp.float32)
    freqr = jnp.exp(colr * jnp.float32(-scale))
    shift = (pid * BLK).astype(jnp.float32) * freqr
    sin_b = jnp.sin(shift)[:1]
    cos_b = jnp.cos(shift)[:1]

    s_a = base_sin[...]
    c_a = base_cos[...]
    out_sin = s_a * cos_b + c_a * sin_b
    out_cos = c_a * cos_b - s_a * sin_b
    bsz = out_ref.shape[0]
    out_ref[:, :, :HALF_DIM] = jnp.broadcast_to(out_sin[None], (bsz, BLK, HALF_DIM))
    out_ref[:, :, HALF_DIM:] = jnp.broadcast_to(out_cos[None], (bsz, BLK, HALF_DIM))


def kernel(input):
    bsz, seqlen = input.shape
    grid = (seqlen // BLK,)
    out = pl.pallas_call(
        _sinusoid_kernel,
        grid=grid,
        out_specs=pl.BlockSpec((bsz, BLK, EMB_DIM), lambda i: (0, i, 0)),
        out_shape=jax.ShapeDtypeStruct((bsz, seqlen, EMB_DIM), input.dtype),
        scratch_shapes=[
            pltpu.VMEM((BLK, HALF_DIM), jnp.float32),
            pltpu.VMEM((BLK, HALF_DIM), jnp.float32),
        ],
    )()
    return out


# speedup vs baseline: 1.0990x; 1.0990x over previous
import math

import jax
import jax.numpy as jnp
from jax.experimental import pallas as pl
from jax.experimental.pallas import tpu as pltpu

EMB_DIM = 1024
HALF_DIM = EMB_DIM // 2
PAD_IDX = 0
BLK = 512


def _sinusoid_kernel(out_ref, base_sin, base_cos):
    pid = pl.program_id(0)
    scale = math.log(10000.0) / (HALF_DIM - 1)

    @pl.when(pid == 0)
    def _init():
        row = jax.lax.broadcasted_iota(jnp.int32, (BLK, HALF_DIM), 0)
        col = jax.lax.broadcasted_iota(jnp.int32, (BLK, HALF_DIM), 1)
        freq = jnp.exp(col.astype(jnp.float32) * jnp.float32(-scale))
        phase = (row.astype(jnp.float32) + jnp.float32(PAD_IDX + 1)) * freq
        base_sin[...] = jnp.sin(phase)
        base_cos[...] = jnp.cos(phase)

    colr = jax.lax.broadcasted_iota(jnp.int32, (8, HALF_DIM), 1).astype(jnp.float32)
    freqr = jnp.exp(colr * jnp.float32(-scale))
    shift = (pid * BLK).astype(jnp.float32) * freqr
    sin_b = jnp.sin(shift)[:1]
    cos_b = jnp.cos(shift)[:1]

    s_a = base_sin[...]
    c_a = base_cos[...]
    out_sin = s_a * cos_b + c_a * sin_b
    out_cos = c_a * cos_b - s_a * sin_b
    bsz = out_ref.shape[0]
    out_ref[:, :, :HALF_DIM] = jnp.broadcast_to(out_sin[None], (bsz, BLK, HALF_DIM))
    out_ref[:, :, HALF_DIM:] = jnp.broadcast_to(out_cos[None], (bsz, BLK, HALF_DIM))


def kernel(input):
    bsz, seqlen = input.shape
    grid = (seqlen // BLK,)
    out = pl.pallas_call(
        _sinusoid_kernel,
        grid=grid,
        out_specs=pl.BlockSpec((bsz, BLK, EMB_DIM), lambda i: (0, i, 0)),
        out_shape=jax.ShapeDtypeStruct((bsz, seqlen, EMB_DIM), input.dtype),
        scratch_shapes=[
            pltpu.VMEM((BLK, HALF_DIM), jnp.float32),
            pltpu.VMEM((BLK, HALF_DIM), jnp.float32),
        ],
    )()
    return out


# speedup vs baseline: 1.1921x; 1.0848x over previous
import math

import jax
import jax.numpy as jnp
from jax.experimental import pallas as pl
from jax.experimental.pallas import tpu as pltpu

EMB_DIM = 1024
HALF_DIM = EMB_DIM // 2
PAD_IDX = 0
BLK = 256


def _sinusoid_kernel(out_ref, base_sin, base_cos):
    pid = pl.program_id(0)
    scale = math.log(10000.0) / (HALF_DIM - 1)

    @pl.when(pid == 0)
    def _init():
        row = jax.lax.broadcasted_iota(jnp.int32, (BLK, HALF_DIM), 0)
        col = jax.lax.broadcasted_iota(jnp.int32, (BLK, HALF_DIM), 1)
        freq = jnp.exp(col.astype(jnp.float32) * jnp.float32(-scale))
        phase = (row.astype(jnp.float32) + jnp.float32(PAD_IDX + 1)) * freq
        base_sin[...] = jnp.sin(phase)
        base_cos[...] = jnp.cos(phase)

    colr = jax.lax.broadcasted_iota(jnp.int32, (8, HALF_DIM), 1).astype(jnp.float32)
    freqr = jnp.exp(colr * jnp.float32(-scale))
    shift = (pid * BLK).astype(jnp.float32) * freqr
    sin_b = jnp.sin(shift)[:1]
    cos_b = jnp.cos(shift)[:1]

    s_a = base_sin[...]
    c_a = base_cos[...]
    out_sin = s_a * cos_b + c_a * sin_b
    out_cos = c_a * cos_b - s_a * sin_b
    block = jnp.concatenate([out_sin, out_cos], axis=1)
    out_ref[...] = jnp.broadcast_to(block[None], out_ref.shape)


def kernel(input):
    bsz, seqlen = input.shape
    grid = (seqlen // BLK,)
    out = pl.pallas_call(
        _sinusoid_kernel,
        grid=grid,
        out_specs=pl.BlockSpec((bsz, BLK, EMB_DIM), lambda i: (0, i, 0)),
        out_shape=jax.ShapeDtypeStruct((bsz, seqlen, EMB_DIM), input.dtype),
        scratch_shapes=[
            pltpu.VMEM((BLK, HALF_DIM), jnp.float32),
            pltpu.VMEM((BLK, HALF_DIM), jnp.float32),
        ],
    )()
    return out
